# baseline (speedup 1.0000x reference)
import functools

import jax
import jax.numpy as jnp
from jax import lax
from jax.experimental import pallas as pl
from jax.experimental.pallas import tpu as pltpu
from jax.experimental.pallas import tpu_sc as plsc

_ROWS = 1_000_000
_D = 64
_B = 16384
_NC = 2
_NS = 16
_NW = _NC * _NS
_BPW = _B // _NW
_CHUNK = 128
_NCHUNK = _BPW // _CHUNK
_L = 16
_Q = 3


def _sc_gather_sorted(table_t, idx_sorted, order3):
    mesh = plsc.VectorSubcoreMesh(core_axis_name="c", subcore_axis_name="s")

    @functools.partial(
        pl.kernel,
        mesh=mesh,
        out_type=jax.ShapeDtypeStruct((_B, 2 * _D), jnp.float32),
        compiler_params=pltpu.CompilerParams(needs_layout_passes=False),
        scratch_types=[
            pltpu.SMEM((_BPW,), jnp.int32),
            pltpu.VMEM((_Q * _D, 2 * _D), jnp.float32),
            pltpu.VMEM((_BPW, 2 * _D), jnp.float32),
            pltpu.VMEM((_NCHUNK, _CHUNK), jnp.int32),
            pltpu.VMEM((_BPW,), jnp.int32),
            pltpu.SemaphoreType.DMA,
            pltpu.SemaphoreType.DMA,
        ],
    )
    def k(table_hbm, idx_hbm, ord_hbm, out_hbm,
          rval_s, win_v, rowbuf_v, ord_v, idx_v, sem, osem):
        wid = lax.axis_index("s") * _NC + lax.axis_index("c")
        base = wid * _BPW
        pltpu.sync_copy(idx_hbm.at[pl.ds(base, _BPW)], idx_v)
        pltpu.sync_copy(ord_hbm.at[wid], ord_v)

        ids16 = lax.iota(jnp.int32, _L)

        def read_idx(l):
            v = idx_v[pl.ds(lax.div(l, _L) * _L, _L)]
            lane = lax.rem(l, _L)
            return jnp.sum(jnp.where(ids16 == lane, v, 0))

        def fire_r(r, slot):
            pltpu.async_copy(
                table_hbm.at[:, pl.ds(pl.multiple_of(r * 128, 128), 128)],
                win_v.at[pl.ds(pl.multiple_of(slot * _D, _D), _D)],
                sem,
            )

        def fire(run_k, slot):
            fire_r(rval_s[run_k], slot)

        def prescan(l, carry):
            nruns, prev = carry
            r = read_idx(l) >> 7
            is_new = jnp.logical_or(l == 0, r != prev)
            nruns = jnp.where(is_new, nruns + 1, nruns)

            @pl.when(is_new)
            def _():
                rval_s[nruns - 1] = r

                @pl.when(nruns <= _Q - 1)
                def _():
                    fire_r(r, nruns - 1)

            return (nruns, r)

        nruns, _ = lax.fori_loop(0, _BPW, prescan, (0, -1))

        def item_body(p, carry):
            runk, prev = carry
            i = read_idx(p)
            r = i >> 7
            is_new = jnp.logical_or(p == 0, r != prev)
            runk = jnp.where(is_new, runk + 1, runk)

            @pl.when(is_new)
            def _():
                pltpu.make_async_copy(
                    table_hbm.at[:, pl.ds(0, 128)],
                    win_v.at[pl.ds(0, _D)],
                    sem,
                ).wait()
                nxt = runk + _Q - 1

                @pl.when(nxt < nruns)
                def _():
                    fire(nxt, lax.rem(nxt, _Q))

            slot = lax.rem(runk, _Q)
            ri = i & 127
            for g in range(_D // _L):
                rows16 = slot * _D + g * _L + ids16
                vals = plsc.load_gather(win_v, [rows16, ids16 * 0 + ri])
                plsc.store_scatter(
                    rowbuf_v, [ids16 * 0 + p, g * _L + ids16], vals
                )

            @pl.when(lax.rem(p + 1, _CHUNK) == 0)
            def _():
                j = lax.div(p + 1, _CHUNK) - 1
                pltpu.async_copy(
                    rowbuf_v.at[
                        pl.ds(pl.multiple_of(j * _CHUNK, _CHUNK), _CHUNK)
                    ],
                    out_hbm.at[ord_v.at[j]],
                    osem,
                )

            return (runk, r)

        lax.fori_loop(0, _BPW, item_body, (-1, -1))

        for j in range(_NCHUNK):
            pltpu.make_async_copy(
                rowbuf_v.at[pl.ds(0, _CHUNK)],
                out_hbm.at[ord_v.at[0]],
                osem,
            ).wait()

    return k(table_t, idx_sorted, order3)


def kernel(param, mesh_indices):
    table_t = param.T
    idx = mesh_indices.astype(jnp.int32)
    iota = lax.iota(jnp.int32, _B)
    idx_sorted, order = lax.sort((idx, iota), num_keys=1)
    order3 = order.reshape(_NW, _NCHUNK, _CHUNK)
    out_pad = _sc_gather_sorted(table_t, idx_sorted, order3)
    return out_pad[:, :_D]


# speedup vs baseline: 1.0421x; 1.0421x over previous
import functools

import jax
import jax.numpy as jnp
from jax import lax
from jax.experimental import pallas as pl
from jax.experimental.pallas import tpu as pltpu
from jax.experimental.pallas import tpu_sc as plsc

_ROWS = 1_000_000
_D = 64
_B = 16384
_NC = 2
_NS = 16
_NW = _NC * _NS
_BPW = _B // _NW
_CHUNK = 128
_NCHUNK = _BPW // _CHUNK
_L = 16
_Q = 3


def _sc_gather_sorted(table_t, idx_sorted, order3):
    mesh = plsc.VectorSubcoreMesh(core_axis_name="c", subcore_axis_name="s")

    @functools.partial(
        pl.kernel,
        mesh=mesh,
        out_type=jax.ShapeDtypeStruct((_B, 2 * _D), jnp.float32),
        compiler_params=pltpu.CompilerParams(needs_layout_passes=False),
        scratch_types=[
            pltpu.SMEM((_BPW,), jnp.int32),
            pltpu.VMEM((_Q * _D, 2 * _D), jnp.float32),
            pltpu.VMEM((_BPW, 2 * _D), jnp.float32),
            pltpu.VMEM((_NCHUNK, _CHUNK), jnp.int32),
            pltpu.VMEM((_BPW,), jnp.int32),
            pltpu.SemaphoreType.DMA,
            pltpu.SemaphoreType.DMA,
        ],
    )
    def k(table_hbm, idx_hbm, ord_hbm, out_hbm,
          rval_s, win_v, rowbuf_v, ord_v, idx_v, sem, osem):
        wid = lax.axis_index("s") * _NC + lax.axis_index("c")
        base = wid * _BPW
        pltpu.sync_copy(idx_hbm.at[pl.ds(base, _BPW)], idx_v)
        pltpu.sync_copy(ord_hbm.at[wid], ord_v)

        ids16 = lax.iota(jnp.int32, _L)

        def read_idx(l):
            v = idx_v[pl.ds(lax.div(l, _L) * _L, _L)]
            lane = lax.rem(l, _L)
            return jnp.sum(jnp.where(ids16 == lane, v, 0))

        def prescan(l, carry):
            nruns, prev = carry
            r = read_idx(l) >> 7
            is_new = jnp.logical_or(l == 0, r != prev)
            nruns = jnp.where(is_new, nruns + 1, nruns)

            @pl.when(is_new)
            def _():
                rval_s[nruns - 1] = r

            return (nruns, r)

        nruns, _ = lax.fori_loop(0, _BPW, prescan, (0, -1))

        def fire(run_k, slot):
            r = rval_s[run_k]
            pltpu.async_copy(
                table_hbm.at[:, pl.ds(pl.multiple_of(r * 128, 128), 128)],
                win_v.at[pl.ds(pl.multiple_of(slot * _D, _D), _D)],
                sem,
            )

        for q in range(_Q - 1):
            @pl.when(q < nruns)
            def _(q=q):
                fire(q, q)

        def item_body(p, carry):
            runk, prev = carry
            i = read_idx(p)
            r = i >> 7
            is_new = jnp.logical_or(p == 0, r != prev)
            runk = jnp.where(is_new, runk + 1, runk)

            @pl.when(is_new)
            def _():
                pltpu.make_async_copy(
                    table_hbm.at[:, pl.ds(0, 128)],
                    win_v.at[pl.ds(0, _D)],
                    sem,
                ).wait()
                nxt = runk + _Q - 1

                @pl.when(nxt < nruns)
                def _():
                    fire(nxt, lax.rem(nxt, _Q))

            slot = lax.rem(runk, _Q)
            ri = i & 127
            for g in range(_D // _L):
                rows16 = slot * _D + g * _L + ids16
                vals = plsc.load_gather(win_v, [rows16, ids16 * 0 + ri])
                plsc.store_scatter(
                    rowbuf_v, [ids16 * 0 + p, g * _L + ids16], vals
                )
            return (runk, r)

        lax.fori_loop(0, _BPW, item_body, (-1, -1))

        outs = []
        for j in range(_NCHUNK):
            outs.append(
                pltpu.async_copy(
                    rowbuf_v.at[pl.ds(j * _CHUNK, _CHUNK)],
                    out_hbm.at[ord_v.at[j]],
                    osem,
                )
            )
        for c in outs:
            c.wait()

    return k(table_t, idx_sorted, order3)


def kernel(param, mesh_indices):
    table_t = param.T
    idx = mesh_indices.astype(jnp.int32)
    iota = lax.iota(jnp.int32, _B)
    idx_sorted, order = lax.sort((idx, iota), num_keys=1)
    order3 = order.reshape(_NW, _NCHUNK, _CHUNK)
    out_pad = _sc_gather_sorted(table_t, idx_sorted, order3)
    return out_pad[:, :_D]


# speedup vs baseline: 1.4539x; 1.3952x over previous
import functools

import jax
import jax.numpy as jnp
from jax import lax
from jax.experimental import pallas as pl
from jax.experimental.pallas import tpu as pltpu
from jax.experimental.pallas import tpu_sc as plsc

_ROWS = 1_000_000
_D = 64
_B = 16384
_NC = 2
_NS = 16
_NW = _NC * _NS
_BPW = _B // _NW
_CHUNK = 128
_NCHUNK = _BPW // _CHUNK
_L = 16
_Q = 6
_RB = 2


def _sc_gather_sorted(table_t, idx_sorted, order3):
    mesh = plsc.VectorSubcoreMesh(core_axis_name="c", subcore_axis_name="s")

    @functools.partial(
        pl.kernel,
        mesh=mesh,
        out_type=jax.ShapeDtypeStruct((_B, 2 * _D), jnp.float32),
        compiler_params=pltpu.CompilerParams(needs_layout_passes=False),
        scratch_types=[
            pltpu.SMEM((_BPW,), jnp.int32),
            pltpu.VMEM((_Q * _D, 2 * _D), jnp.float32),
            pltpu.VMEM((_RB * _CHUNK, 2 * _D), jnp.float32),
            pltpu.VMEM((_NCHUNK, _CHUNK), jnp.int32),
            pltpu.VMEM((_BPW,), jnp.int32),
            pltpu.SemaphoreType.DMA,
            pltpu.SemaphoreType.DMA,
        ],
    )
    def k(table_hbm, idx_hbm, ord_hbm, out_hbm,
          rval_s, win_v, rowbuf_v, ord_v, idx_v, sem, osem):
        wid = lax.axis_index("s") * _NC + lax.axis_index("c")
        base = wid * _BPW
        pltpu.sync_copy(idx_hbm.at[pl.ds(base, _BPW)], idx_v)
        pltpu.sync_copy(ord_hbm.at[wid], ord_v)

        ids16 = lax.iota(jnp.int32, _L)

        def read_idx(l):
            v = idx_v[pl.ds(lax.div(l, _L) * _L, _L)]
            lane = lax.rem(l, _L)
            return jnp.sum(jnp.where(ids16 == lane, v, 0))

        def prescan(l, carry):
            nruns, prev = carry
            r = read_idx(l) >> 7
            is_new = jnp.logical_or(l == 0, r != prev)
            nruns = jnp.where(is_new, nruns + 1, nruns)

            @pl.when(is_new)
            def _():
                rval_s[nruns - 1] = r

            return (nruns, r)

        nruns, _ = lax.fori_loop(0, _BPW, prescan, (0, -1))

        def fire(run_k, slot):
            r = rval_s[run_k]
            pltpu.async_copy(
                table_hbm.at[:, pl.ds(pl.multiple_of(r * 128, 128), 128)],
                win_v.at[pl.ds(pl.multiple_of(slot * _D, _D), _D)],
                sem,
            )

        for q in range(_Q - 1):
            @pl.when(q < nruns)
            def _(q=q):
                fire(q, q)

        def item_body(p, carry):
            runk, prev = carry
            i = read_idx(p)
            r = i >> 7
            is_new = jnp.logical_or(p == 0, r != prev)
            runk = jnp.where(is_new, runk + 1, runk)

            @pl.when(is_new)
            def _():
                pltpu.make_async_copy(
                    table_hbm.at[:, pl.ds(0, 128)],
                    win_v.at[pl.ds(0, _D)],
                    sem,
                ).wait()
                nxt = runk + _Q - 1

                @pl.when(nxt < nruns)
                def _():
                    fire(nxt, lax.rem(nxt, _Q))

            @pl.when(jnp.logical_and(lax.rem(p, _CHUNK) == 0,
                                     p >= _RB * _CHUNK))
            def _():
                pltpu.make_async_copy(
                    rowbuf_v.at[pl.ds(0, _CHUNK)],
                    out_hbm.at[ord_v.at[0]],
                    osem,
                ).wait()

            slot = lax.rem(runk, _Q)
            rbrow = lax.rem(lax.div(p, _CHUNK), _RB) * _CHUNK + lax.rem(p, _CHUNK)
            ri = i & 127
            for g in range(_D // _L):
                rows16 = slot * _D + g * _L + ids16
                vals = plsc.load_gather(win_v, [rows16, ids16 * 0 + ri])
                plsc.store_scatter(
                    rowbuf_v, [ids16 * 0 + rbrow, g * _L + ids16], vals
                )

            @pl.when(lax.rem(p + 1, _CHUNK) == 0)
            def _():
                j = lax.div(p + 1, _CHUNK) - 1
                rbs = lax.rem(j, _RB)
                pltpu.async_copy(
                    rowbuf_v.at[
                        pl.ds(pl.multiple_of(rbs * _CHUNK, _CHUNK), _CHUNK)
                    ],
                    out_hbm.at[ord_v.at[j]],
                    osem,
                )

            return (runk, r)

        lax.fori_loop(0, _BPW, item_body, (-1, -1))

        for j in range(_RB):
            pltpu.make_async_copy(
                rowbuf_v.at[pl.ds(0, _CHUNK)],
                out_hbm.at[ord_v.at[0]],
                osem,
            ).wait()

    return k(table_t, idx_sorted, order3)


def kernel(param, mesh_indices):
    table_t = param.T
    idx = mesh_indices.astype(jnp.int32)
    iota = lax.iota(jnp.int32, _B)
    idx_sorted, order = lax.sort((idx, iota), num_keys=1)
    order3 = order.reshape(_NW, _NCHUNK, _CHUNK)
    out_pad = _sc_gather_sorted(table_t, idx_sorted, order3)
    return out_pad[:, :_D]


# speedup vs baseline: 1.4588x; 1.0033x over previous
import functools

import jax
import jax.numpy as jnp
from jax import lax
from jax.experimental import pallas as pl
from jax.experimental.pallas import tpu as pltpu
from jax.experimental.pallas import tpu_sc as plsc

_ROWS = 1_000_000
_D = 64
_B = 16384
_NC = 2
_NS = 16
_NW = _NC * _NS
_BPW = _B // _NW
_CHUNK = 128
_NCHUNK = _BPW // _CHUNK
_L = 16
_Q = 7
_RB = 2


def _sc_gather_sorted(table_t, idx_sorted, order3):
    mesh = plsc.VectorSubcoreMesh(core_axis_name="c", subcore_axis_name="s")

    @functools.partial(
        pl.kernel,
        mesh=mesh,
        out_type=jax.ShapeDtypeStruct((_B, 2 * _D), jnp.float32),
        compiler_params=pltpu.CompilerParams(needs_layout_passes=False),
        scratch_types=[
            pltpu.SMEM((_BPW,), jnp.int32),
            pltpu.VMEM((_Q * _D, 2 * _D), jnp.float32),
            pltpu.VMEM((_RB * _CHUNK, 2 * _D), jnp.float32),
            pltpu.VMEM((_NCHUNK, _CHUNK), jnp.int32),
            pltpu.VMEM((_BPW,), jnp.int32),
            pltpu.SemaphoreType.DMA,
            pltpu.SemaphoreType.DMA,
        ],
    )
    def k(table_hbm, idx_hbm, ord_hbm, out_hbm,
          rval_s, win_v, rowbuf_v, ord_v, idx_v, sem, osem):
        wid = lax.axis_index("s") * _NC + lax.axis_index("c")
        base = wid * _BPW
        pltpu.sync_copy(idx_hbm.at[pl.ds(base, _BPW)], idx_v)
        pltpu.sync_copy(ord_hbm.at[wid], ord_v)

        ids16 = lax.iota(jnp.int32, _L)

        def read_idx(l):
            v = idx_v[pl.ds(lax.div(l, _L) * _L, _L)]
            lane = lax.rem(l, _L)
            return jnp.sum(jnp.where(ids16 == lane, v, 0))

        def prescan(l, carry):
            nruns, prev = carry
            r = read_idx(l) >> 7
            is_new = jnp.logical_or(l == 0, r != prev)
            nruns = jnp.where(is_new, nruns + 1, nruns)

            @pl.when(is_new)
            def _():
                rval_s[nruns - 1] = r

            return (nruns, r)

        nruns, _ = lax.fori_loop(0, _BPW, prescan, (0, -1))

        def fire(run_k, slot):
            r = rval_s[run_k]
            pltpu.async_copy(
                table_hbm.at[:, pl.ds(pl.multiple_of(r * 128, 128), 128)],
                win_v.at[pl.ds(pl.multiple_of(slot * _D, _D), _D)],
                sem,
            )

        for q in range(_Q - 1):
            @pl.when(q < nruns)
            def _(q=q):
                fire(q, q)

        def item_body(p, carry):
            runk, prev = carry
            i = read_idx(p)
            r = i >> 7
            is_new = jnp.logical_or(p == 0, r != prev)
            runk = jnp.where(is_new, runk + 1, runk)

            @pl.when(is_new)
            def _():
                pltpu.make_async_copy(
                    table_hbm.at[:, pl.ds(0, 128)],
                    win_v.at[pl.ds(0, _D)],
                    sem,
                ).wait()
                nxt = runk + _Q - 1

                @pl.when(nxt < nruns)
                def _():
                    fire(nxt, lax.rem(nxt, _Q))

            @pl.when(jnp.logical_and(lax.rem(p, _CHUNK) == 0,
                                     p >= _RB * _CHUNK))
            def _():
                pltpu.make_async_copy(
                    rowbuf_v.at[pl.ds(0, _CHUNK)],
                    out_hbm.at[ord_v.at[0]],
                    osem,
                ).wait()

            slot = lax.rem(runk, _Q)
            rbrow = lax.rem(lax.div(p, _CHUNK), _RB) * _CHUNK + lax.rem(p, _CHUNK)
            ri = i & 127
            for g in range(_D // _L):
                rows16 = slot * _D + g * _L + ids16
                vals = plsc.load_gather(win_v, [rows16, ids16 * 0 + ri])
                plsc.store_scatter(
                    rowbuf_v, [ids16 * 0 + rbrow, g * _L + ids16], vals
                )

            @pl.when(lax.rem(p + 1, _CHUNK) == 0)
            def _():
                j = lax.div(p + 1, _CHUNK) - 1
                rbs = lax.rem(j, _RB)
                pltpu.async_copy(
                    rowbuf_v.at[
                        pl.ds(pl.multiple_of(rbs * _CHUNK, _CHUNK), _CHUNK)
                    ],
                    out_hbm.at[ord_v.at[j]],
                    osem,
                )

            return (runk, r)

        lax.fori_loop(0, _BPW, item_body, (-1, -1))

        for j in range(_RB):
            pltpu.make_async_copy(
                rowbuf_v.at[pl.ds(0, _CHUNK)],
                out_hbm.at[ord_v.at[0]],
                osem,
            ).wait()

    return k(table_t, idx_sorted, order3)


def kernel(param, mesh_indices):
    table_t = param.T
    idx = mesh_indices.astype(jnp.int32)
    iota = lax.iota(jnp.int32, _B)
    idx_sorted, order = lax.sort((idx, iota), num_keys=1)
    order3 = order.reshape(_NW, _NCHUNK, _CHUNK)
    out_pad = _sc_gather_sorted(table_t, idx_sorted, order3)
    return out_pad[:, :_D]
